# baseline (speedup 1.0000x reference)
import functools

import jax
import jax.numpy as jnp
from jax import lax
from jax.experimental import pallas as pl
from jax.experimental.pallas import tpu as pltpu
from jax.experimental.pallas import tpu_sc as plsc

B = 16384
EMB = 32
NC = 2
NS = 16
NW = NC * NS
BPW = B // NW
CHUNK = 128
NCHUNK = BPW // CHUNK

TABLE_OF = (0, 1, 2, 3, 3, 3, 2, 2, 2)


def _sc_gather_body(user_t, item_t, cate_t, hist_t, idx_hbm, out_hbm,
                    idx_v, rows_v, sem):
  c = lax.axis_index("c")
  s = lax.axis_index("s")
  wid = s * NC + c
  base = wid * BPW
  tables = (user_t, item_t, cate_t, hist_t)
  for k in range(9):
    tab = tables[TABLE_OF[k]]
    pltpu.sync_copy(idx_hbm.at[k * NW + wid], idx_v)
    cps = []
    for j in range(NCHUNK):
      cps.append(
          pltpu.async_copy(tab.at[idx_v.at[j]],
                           rows_v.at[pl.ds(j * CHUNK, CHUNK)], sem))
    for cp in cps:
      cp.wait()
    pltpu.sync_copy(rows_v, out_hbm.at[k, pl.ds(base, BPW)])


@functools.partial(jax.jit, static_argnames=())
def _sc_gather(user_emb, item_emb, cate_emb, hist_emb, idx9):
  mesh = plsc.VectorSubcoreMesh(core_axis_name="c", subcore_axis_name="s")
  k = pl.kernel(
      _sc_gather_body,
      out_type=jax.ShapeDtypeStruct((9, B, EMB), jnp.float32),
      mesh=mesh,
      scratch_types=[
          pltpu.VMEM((NCHUNK, CHUNK), jnp.int32),
          pltpu.VMEM((BPW, EMB), jnp.float32),
          pltpu.SemaphoreType.DMA,
      ],
      compiler_params=pltpu.CompilerParams(use_tc_tiling_on_sc=False),
  )
  return k(user_emb, item_emb, cate_emb, hist_emb, idx9)


BC = 1024


def _mlp_body(g_ref, n4_ref, w1, b1, w2, b2, w3, b3, w4, b4, out_ref):
  parts = [g_ref[k] for k in range(9)]
  parts.append(n4_ref[...])
  x = jnp.concatenate(parts, axis=1)
  h = jnp.maximum(
      jnp.dot(x, w1[...], preferred_element_type=jnp.float32) + b1[...], 0.0)
  h = jnp.maximum(
      jnp.dot(h, w2[...], preferred_element_type=jnp.float32) + b2[...], 0.0)
  h = jnp.maximum(
      jnp.dot(h, w3[...], preferred_element_type=jnp.float32) + b3[...], 0.0)
  z = jnp.dot(h, w4[...], preferred_element_type=jnp.float32) + b4[...]
  out_ref[...] = 1.0 / (1.0 + jnp.exp(-z))


def _mlp(gath, n4, W1, b1, W2, b2, W3, b3, W4, b4):
  full = lambda shape: pl.BlockSpec(shape, lambda i: (0,) * len(shape))
  return pl.pallas_call(
      _mlp_body,
      grid=(B // BC,),
      in_specs=[
          pl.BlockSpec((9, BC, EMB), lambda i: (0, i, 0)),
          pl.BlockSpec((BC, 4), lambda i: (i, 0)),
          full(W1.shape), full((1, 512)),
          full(W2.shape), full((1, 256)),
          full(W3.shape), full((1, 128)),
          full(W4.shape), full((1, 1)),
      ],
      out_specs=pl.BlockSpec((BC, 1), lambda i: (i, 0)),
      out_shape=jax.ShapeDtypeStruct((B, 1), jnp.float32),
  )(gath, n4, W1, b1, W2, b2, W3, b3, W4, b4)


def kernel(u, i, c, i1, i2, i3, c1, c2, c3, nv, nf, nc, nb,
           user_emb, item_emb, cate_emb, hist_emb,
           W1, b1, W2, b2, W3, b3, W4, b4):
  idx9 = jnp.stack([u, i, c, i1, i2, i3, c1, c2, c3]).astype(jnp.int32)
  idx9 = idx9.reshape(9 * NW, NCHUNK, CHUNK)
  gath = _sc_gather(user_emb, item_emb, cate_emb, hist_emb, idx9)
  n4 = jnp.stack([nv, nf, nc, nb], axis=1)
  out = _mlp(gath, n4,
             W1, b1.reshape(1, -1), W2, b2.reshape(1, -1),
             W3, b3.reshape(1, -1), W4, b4.reshape(1, -1))
  return out[:, 0]
